# baseline (speedup 1.0000x reference)
import functools
import jax
import jax.numpy as jnp
from jax import lax
from jax.experimental import pallas as pl
from jax.experimental.pallas import tpu as pltpu
from jax.experimental.pallas import tpu_sc as plsc

PAD = 1
L = 16
CH = 32
SJ = 256
SC_BATCHES = 1


def _tc_body(inpT_ref, w_ref, row1_ref, out_ref):
    w = w_ref[...]
    row1 = row1_ref[...]
    bsz = inpT_ref.shape[1]
    for b in range(bsz):
        mask = inpT_ref[:, b : b + 1] != PAD
        out_ref[b] = jnp.where(mask, w, row1)


def _tc_call(input, weights):
    bsz, seq_len = input.shape
    d = weights.shape[1]
    inpT = input.T
    wslab = jax.lax.slice(weights, (2, 0), (2 + seq_len, d))
    row1 = jax.lax.slice(weights, (PAD, 0), (PAD + 1, d))
    grid = (seq_len // SJ,)
    return pl.pallas_call(
        _tc_body,
        grid=grid,
        in_specs=[
            pl.BlockSpec((SJ, bsz), lambda j: (j, 0)),
            pl.BlockSpec((SJ, d), lambda j: (j, 0)),
            pl.BlockSpec((1, d), lambda j: (0, 0)),
        ],
        out_specs=pl.BlockSpec((bsz, SJ, d), lambda j: (0, j, 0)),
        out_shape=jax.ShapeDtypeStruct((bsz, seq_len, d), jnp.float32),
    )(inpT, wslab, row1)


def _make_sc(total, seq_len, d):
    info = plsc.get_sparse_core_info()
    nw = info.num_cores * info.num_subcores
    rows_w = total // nw
    nch = rows_w // CH
    assert seq_len % rows_w == 0 and rows_w % CH == 0 and total % nw == 0
    mesh = plsc.VectorSubcoreMesh(core_axis_name="c", subcore_axis_name="s")
    nc = info.num_cores

    @functools.partial(
        pl.kernel,
        mesh=mesh,
        out_type=jax.ShapeDtypeStruct((total, d), jnp.float32),
        scratch_types=[
            pltpu.VMEM((rows_w,), jnp.int32),
            pltpu.VMEM((nch, CH), jnp.int32),
            pltpu.VMEM((CH, d), jnp.float32),
            pltpu.VMEM((CH, d), jnp.float32),
            pltpu.SemaphoreType.DMA,
            pltpu.SemaphoreType.DMA,
            pltpu.SemaphoreType.DMA,
            pltpu.SemaphoreType.DMA,
        ],
    )
    def k(inp_hbm, table_hbm, out_hbm, tok_v, idx_v, buf0, buf1, g0, g1, s0, s1):
        wid = lax.axis_index("s") * nc + lax.axis_index("c")
        base = wid * rows_w
        jbase = lax.rem(base, seq_len) + 2

        pltpu.sync_copy(inp_hbm.at[pl.ds(base, rows_w)], tok_v)

        lane = jnp.arange(L, dtype=jnp.int32)
        for ch in range(nch):
            for v in range(CH // L):
                o = ch * CH + v * L
                tok = tok_v[pl.ds(o, L)]
                pos = lane + (jbase + o)
                idx_v[ch, pl.ds(v * L, L)] = jnp.where(tok != PAD, pos, PAD)

        bufs = (buf0, buf1)
        gsems = (g0, g1)
        ssems = (s0, s1)

        pltpu.async_copy(table_hbm.at[idx_v.at[0]], bufs[0], gsems[0])
        for ch in range(nch):
            p = ch % 2
            q = 1 - p
            pltpu.make_async_copy(table_hbm.at[idx_v.at[ch]], bufs[p], gsems[p]).wait()
            if ch + 1 < nch:
                if ch >= 1:
                    pltpu.make_async_copy(
                        bufs[q], out_hbm.at[pl.ds(base + (ch - 1) * CH, CH)], ssems[q]
                    ).wait()
                pltpu.async_copy(table_hbm.at[idx_v.at[ch + 1]], bufs[q], gsems[q])
            pltpu.async_copy(bufs[p], out_hbm.at[pl.ds(base + ch * CH, CH)], ssems[p])
        for ch in (nch - 2, nch - 1):
            p = ch % 2
            pltpu.make_async_copy(
                bufs[p], out_hbm.at[pl.ds(base + ch * CH, CH)], ssems[p]
            ).wait()

    return k


def kernel(input, weights):
    bsz, seq_len = input.shape
    d = weights.shape[1]
    nb = bsz - SC_BATCHES
    sc = _make_sc(SC_BATCHES * seq_len, seq_len, d)
    sc_out = sc(input[nb:].reshape(-1), weights)
    tc_out = _tc_call(input[:nb], weights)
    return jnp.concatenate(
        [tc_out, sc_out.reshape(SC_BATCHES, seq_len, d)], axis=0
    )


# speedup vs baseline: 1.8639x; 1.8639x over previous
import functools
import jax
import jax.numpy as jnp
from jax import lax
from jax.experimental import pallas as pl
from jax.experimental.pallas import tpu as pltpu
from jax.experimental.pallas import tpu_sc as plsc

PAD = 1
L = 16
CH = 32


def _make_sc(bsz, seq_len, d):
    info = plsc.get_sparse_core_info()
    nc = info.num_cores
    nw = nc * info.num_subcores
    js_w = seq_len // nw
    nch = js_w // CH
    assert seq_len % nw == 0 and js_w % CH == 0
    mesh = plsc.VectorSubcoreMesh(core_axis_name="c", subcore_axis_name="s")

    @functools.partial(
        pl.kernel,
        mesh=mesh,
        out_type=jax.ShapeDtypeStruct((bsz * seq_len, d), jnp.float32),
        scratch_types=[
            pltpu.VMEM((bsz, js_w), jnp.int32),
            pltpu.VMEM((CH, d), jnp.float32),
            pltpu.VMEM((CH, d), jnp.float32),
            pltpu.VMEM((nch, CH), jnp.int32),
            pltpu.VMEM((CH,), jnp.int32),
            pltpu.VMEM((CH, d), jnp.float32),
            pltpu.SemaphoreType.DMA,
            pltpu.SemaphoreType.DMA,
            pltpu.SemaphoreType.DMA,
            pltpu.SemaphoreType.DMA,
            pltpu.SemaphoreType.DMA,
            pltpu.SemaphoreType.DMA,
        ],
    )
    def k(inp_hbm, table_hbm, out_hbm, tok_v, buf0, buf1, iidx, fidx, fbuf,
          g0, g1, s0, s1, fg, fs):
        wid = lax.axis_index("s") * nc + lax.axis_index("c")
        j0 = pl.multiple_of(wid * js_w, js_w)

        for b in range(bsz):
            pltpu.sync_copy(
                inp_hbm.at[pl.ds(b * seq_len + j0, js_w)], tok_v.at[b]
            )

        bufs = (buf0, buf1)
        gsems = (g0, g1)
        ssems = (s0, s1)
        lane = jnp.arange(L, dtype=jnp.int32)

        for ch in range(nch):
            for v in range(CH // L):
                iidx[ch, pl.ds(v * L, L)] = lane + (j0 + ch * CH + v * L + 2)

        has_pad = []
        for b in range(bsz):
            acc = jnp.where(tok_v[b, pl.ds(0, L)] == PAD, 1, 0)
            for v in range(1, js_w // L):
                tok = tok_v[b, pl.ds(v * L, L)]
                acc = acc | jnp.where(tok == PAD, 1, 0)
            s = acc[0]
            for i in range(1, L):
                s = s | acc[i]
            has_pad.append(s > 0)

        def clean_gather(ch, p):
            return pltpu.make_async_copy(
                table_hbm.at[iidx.at[ch]], bufs[p], gsems[p]
            )

        def out_slice(b, ch):
            start = pl.multiple_of(b * seq_len + j0 + ch * CH, 8)
            return out_hbm.at[pl.ds(start, CH)]

        clean_gather(0, 0).start()
        for ch in range(nch):
            p = ch % 2
            q = 1 - p
            clean_gather(ch, p).wait()
            if ch + 1 < nch:
                if ch >= 1:
                    for b in range(bsz):
                        pltpu.make_async_copy(
                            bufs[q], out_slice(b, ch - 1), ssems[q]
                        ).wait()
                clean_gather(ch + 1, q).start()
            for b in range(bsz):
                pltpu.make_async_copy(bufs[p], out_slice(b, ch), ssems[p]).start()
        for ch in (nch - 2, nch - 1):
            p = ch % 2
            for b in range(bsz):
                pltpu.make_async_copy(bufs[p], out_slice(b, ch), ssems[p]).wait()

        for b in range(bsz):

            @pl.when(has_pad[b])
            def _fixup(b=b):
                for ch in range(nch):
                    for v in range(CH // L):
                        tok = tok_v[b, pl.ds(ch * CH + v * L, L)]
                        pos = lane + (j0 + ch * CH + v * L + 2)
                        fidx[pl.ds(v * L, L)] = jnp.where(tok != PAD, pos, PAD)
                    pltpu.make_async_copy(table_hbm.at[fidx], fbuf, fg).start()
                    pltpu.make_async_copy(table_hbm.at[fidx], fbuf, fg).wait()
                    pltpu.make_async_copy(fbuf, out_slice(b, ch), fs).start()
                    pltpu.make_async_copy(fbuf, out_slice(b, ch), fs).wait()

    return k


def kernel(input, weights):
    bsz, seq_len = input.shape
    d = weights.shape[1]
    k = _make_sc(bsz, seq_len, d)
    out = k(input.reshape(-1), weights)
    return out.reshape(bsz, seq_len, d)


# speedup vs baseline: 1.8708x; 1.0037x over previous
import functools
import jax
import jax.numpy as jnp
from jax import lax
from jax.experimental import pallas as pl
from jax.experimental.pallas import tpu as pltpu
from jax.experimental.pallas import tpu_sc as plsc

PAD = 1
L = 16
CH = 32


def _make_sc(bsz, seq_len, d):
    info = plsc.get_sparse_core_info()
    nc = info.num_cores
    nw = nc * info.num_subcores
    js_w = seq_len // nw
    nch = js_w // CH
    assert seq_len % nw == 0 and js_w % CH == 0
    mesh = plsc.VectorSubcoreMesh(core_axis_name="c", subcore_axis_name="s")

    @functools.partial(
        pl.kernel,
        mesh=mesh,
        out_type=jax.ShapeDtypeStruct((bsz * seq_len, d), jnp.float32),
        scratch_types=[
            pltpu.VMEM((bsz, js_w), jnp.int32),
            pltpu.VMEM((CH, d), jnp.float32),
            pltpu.VMEM((CH, d), jnp.float32),
            pltpu.VMEM((CH, d), jnp.float32),
            pltpu.VMEM((nch, CH), jnp.int32),
            pltpu.VMEM((CH,), jnp.int32),
            pltpu.SemaphoreType.DMA,
            pltpu.SemaphoreType.DMA,
            pltpu.SemaphoreType.DMA,
            pltpu.SemaphoreType.DMA,
            pltpu.SemaphoreType.DMA,
            pltpu.SemaphoreType.DMA,
        ],
    )
    def k(inp_hbm, table_hbm, out_hbm, tok_v, buf0, buf1, buf2, iidx, fidx,
          g0, g1, g2, s0, s1, s2):
        wid = lax.axis_index("s") * nc + lax.axis_index("c")
        j0 = pl.multiple_of(wid * js_w, js_w)

        for b in range(bsz):
            pltpu.sync_copy(
                inp_hbm.at[pl.ds(b * seq_len + j0, js_w)], tok_v.at[b]
            )

        bufs = (buf0, buf1, buf2)
        gsems = (g0, g1, g2)
        ssems = (s0, s1, s2)
        nbuf = len(bufs)
        lane = jnp.arange(L, dtype=jnp.int32)

        for ch in range(nch):
            for v in range(CH // L):
                iidx[ch, pl.ds(v * L, L)] = lane + (j0 + ch * CH + v * L + 2)

        has_pad = []
        for b in range(bsz):
            acc = jnp.where(tok_v[b, pl.ds(0, L)] == PAD, 1, 0)
            for v in range(1, js_w // L):
                tok = tok_v[b, pl.ds(v * L, L)]
                acc = acc | jnp.where(tok == PAD, 1, 0)
            s = acc[0]
            for i in range(1, L):
                s = s | acc[i]
            has_pad.append(s > 0)

        def clean_gather(ch, p):
            return pltpu.make_async_copy(
                table_hbm.at[iidx.at[ch]], bufs[p], gsems[p]
            )

        def out_slice(b, ch):
            start = pl.multiple_of(b * seq_len + j0 + ch * CH, 8)
            return out_hbm.at[pl.ds(start, CH)]

        clean_gather(0, 0).start()
        for ch in range(nch):
            p = ch % nbuf
            clean_gather(ch, p).wait()
            if ch + 1 < nch:
                q = (ch + 1) % nbuf
                if ch >= nbuf - 1:
                    for b in range(bsz):
                        pltpu.make_async_copy(
                            bufs[q], out_slice(b, ch + 1 - nbuf), ssems[q]
                        ).wait()
                clean_gather(ch + 1, q).start()
            for b in range(bsz):
                pltpu.make_async_copy(bufs[p], out_slice(b, ch), ssems[p]).start()
        for ch in range(nch - nbuf, nch):
            p = ch % nbuf
            for b in range(bsz):
                pltpu.make_async_copy(bufs[p], out_slice(b, ch), ssems[p]).wait()

        for b in range(bsz):

            @pl.when(has_pad[b])
            def _fixup(b=b):
                for ch in range(nch):
                    for v in range(CH // L):
                        tok = tok_v[b, pl.ds(ch * CH + v * L, L)]
                        pos = lane + (j0 + ch * CH + v * L + 2)
                        fidx[pl.ds(v * L, L)] = jnp.where(tok != PAD, pos, PAD)
                    pltpu.make_async_copy(table_hbm.at[fidx], buf0, g0).start()
                    pltpu.make_async_copy(table_hbm.at[fidx], buf0, g0).wait()
                    pltpu.make_async_copy(buf0, out_slice(b, ch), s0).start()
                    pltpu.make_async_copy(buf0, out_slice(b, ch), s0).wait()

    return k


def kernel(input, weights):
    bsz, seq_len = input.shape
    d = weights.shape[1]
    k = _make_sc(bsz, seq_len, d)
    out = k(input.reshape(-1), weights)
    return out.reshape(bsz, seq_len, d)
